# baseline (speedup 1.0000x reference)
import functools

import jax
import jax.numpy as jnp
from jax import lax
from jax.experimental import pallas as pl
from jax.experimental.pallas import tpu as pltpu
from jax.experimental.pallas import tpu_sc as plsc

NUM_CORES = 2
NUM_SUBCORES = 16
NW = NUM_CORES * NUM_SUBCORES

D_MODEL = 64
D_PAD = 128
VOCAB = 1000000
BATCH = 4096
SEQ = 50
TOTAL = BATCH * SEQ

LANES = 16

CCOLS = 512
HCOLS = CCOLS // 2
MAIN_CHUNKS = 61
MAIN_COLS = MAIN_CHUNKS * CCOLS * NW
MINI_START = MAIN_COLS + CCOLS
OUT_WORDS = HCOLS * D_PAD

CHUNK = 128
NCHUNK = TOTAL // NW // CHUNK
NBUF = 5


def _iota16():
    return jnp.arange(LANES, dtype=jnp.int32)


def _make_retile():
    mesh = plsc.VectorSubcoreMesh(core_axis_name="c", subcore_axis_name="s")

    @functools.partial(
        pl.kernel,
        mesh=mesh,
        out_type=jax.ShapeDtypeStruct((VOCAB * D_PAD,), jnp.float32),
        scratch_types=[
            pltpu.VMEM((2, 8, 8, CCOLS), jnp.float32),
            pltpu.VMEM((OUT_WORDS,), jnp.float32),
            pltpu.VMEM((HCOLS,), jnp.int32),
            pltpu.SemaphoreType.DMA((2,)),
        ],
        compiler_params=pltpu.CompilerParams(needs_layout_passes=False),
    )
    def retile_kernel(tblT_hbm, big_hbm, in_v, out_v, fvec_v, sems):
        wid = lax.axis_index("s") * NUM_CORES + lax.axis_index("c")
        col0 = wid * (MAIN_CHUNKS * CCOLS)

        for cg in range(HCOLS // LANES):
            c = _iota16() + (LANES * cg)
            fvec_v[pl.ds(LANES * cg, LANES)] = c << 7

        def load_chunk(buf, cstart):
            for k in range(8):
                pltpu.make_async_copy(
                    tblT_hbm.at[pl.ds(8 * k, 8), pl.ds(cstart, CCOLS)],
                    in_v.at[buf, k],
                    sems.at[buf],
                ).start()

        def wait_chunk(buf, cstart):
            for k in range(8):
                pltpu.make_async_copy(
                    tblT_hbm.at[pl.ds(8 * k, 8), pl.ds(cstart, CCOLS)],
                    in_v.at[buf, k],
                    sems.at[buf],
                ).wait()

        def extract_half(buf, h):
            def cgbody(cg, _):
                coff = pl.multiple_of(LANES * cg, LANES)
                fv = fvec_v[pl.ds(coff, LANES)]
                for k in range(8):
                    for dd in range(8):
                        x = in_v[buf, k, dd, pl.ds(coff + h * HCOLS, LANES)]
                        plsc.store_scatter(out_v, [fv + (8 * k + dd)], x)
                return _
            lax.fori_loop(0, HCOLS // LANES, cgbody, None)

        def flush(hstart):
            pltpu.sync_copy(
                out_v,
                big_hbm.at[pl.ds(pl.multiple_of(hstart * D_PAD, OUT_WORDS),
                                 OUT_WORDS)],
            )

        load_chunk(0, col0)

        def step(g, _):
            buf = lax.rem(g, 2)
            cstart = pl.multiple_of(col0 + g * CCOLS, CCOLS)
            wait_chunk(buf, cstart)

            @pl.when(g + 1 < MAIN_CHUNKS)
            def _():
                load_chunk(1 - buf, cstart + CCOLS)

            for h in range(2):
                extract_half(buf, h)
                flush(cstart + h * HCOLS)
            return _

        lax.fori_loop(0, MAIN_CHUNKS, step, None)

        @pl.when(wid == NW - 1)
        def _tail():
            load_chunk(0, MAIN_COLS)
            wait_chunk(0, MAIN_COLS)
            for h in range(2):
                extract_half(0, h)
                flush(MAIN_COLS + h * HCOLS)

    return retile_kernel


def _make_gather():
    mesh = plsc.VectorSubcoreMesh(core_axis_name="c", subcore_axis_name="s")

    @functools.partial(
        pl.kernel,
        mesh=mesh,
        out_type=jax.ShapeDtypeStruct((TOTAL, D_PAD), jnp.float32),
        scratch_types=[
            pltpu.VMEM((NCHUNK, CHUNK), jnp.int32),
            pltpu.VMEM((NBUF, CHUNK, D_PAD), jnp.float32),
            pltpu.SemaphoreType.DMA((NBUF,)),
        ],
    )
    def gather_kernel(big_hbm, idx_hbm, out_hbm, idx_v, rows_v, sems):
        wid = lax.axis_index("s") * NUM_CORES + lax.axis_index("c")
        base = wid * (NCHUNK * CHUNK)

        pltpu.sync_copy(idx_hbm.at[wid], idx_v)

        for b in range(NBUF):
            pltpu.make_async_copy(
                big_hbm.at[idx_v.at[b]], rows_v.at[b], sems.at[b]
            ).start()

        def step(g, _):
            for b in range(NBUF):
                j = g * NBUF + b
                pltpu.make_async_copy(
                    big_hbm.at[idx_v.at[j]], rows_v.at[b], sems.at[b]
                ).wait()
                pltpu.sync_copy(
                    rows_v.at[b], out_hbm.at[pl.ds(base + j * CHUNK, CHUNK)]
                )
                nxt = j + NBUF

                @pl.when(nxt < NCHUNK)
                def _():
                    pltpu.make_async_copy(
                        big_hbm.at[idx_v.at[nxt]], rows_v.at[b], sems.at[b]
                    ).start()

            return _

        lax.fori_loop(0, NCHUNK // NBUF, step, None)

    return gather_kernel


_retile = _make_retile()
_gather = _make_gather()


def kernel(token_ids, table):
    tblT = table.T
    idx3 = token_ids.astype(jnp.int32).reshape(NW, NCHUNK, CHUNK)
    big_flat = _retile(tblT)
    tail = jnp.pad(table[MINI_START:], ((0, 0), (0, D_PAD - D_MODEL)))
    big_flat = lax.dynamic_update_slice(
        big_flat, tail.reshape(-1), (MINI_START * D_PAD,)
    )
    big = big_flat.reshape(VOCAB, D_PAD)
    out = _gather(big, idx3)
    return out[:, :D_MODEL].reshape(BATCH, SEQ, D_MODEL)


# speedup vs baseline: 1.0005x; 1.0005x over previous
import functools

import jax
import jax.numpy as jnp
from jax import lax
from jax.experimental import pallas as pl
from jax.experimental.pallas import tpu as pltpu
from jax.experimental.pallas import tpu_sc as plsc

NUM_CORES = 2
NUM_SUBCORES = 16
NW = NUM_CORES * NUM_SUBCORES

D_MODEL = 64
D_PAD = 128
VOCAB = 1000000
BATCH = 4096
SEQ = 50
TOTAL = BATCH * SEQ

LANES = 16

CCOLS = 256
HCOLS = CCOLS // 2
MAIN_CHUNKS = 122
MAIN_COLS = MAIN_CHUNKS * CCOLS * NW
MINI_START = MAIN_COLS + 2 * CCOLS
OUT_WORDS = HCOLS * D_PAD

CHUNK = 128
NCHUNK = TOTAL // NW // CHUNK
NBUF = 5


def _iota16():
    return jnp.arange(LANES, dtype=jnp.int32)


def _make_retile():
    mesh = plsc.VectorSubcoreMesh(core_axis_name="c", subcore_axis_name="s")

    @functools.partial(
        pl.kernel,
        mesh=mesh,
        out_type=jax.ShapeDtypeStruct((VOCAB, D_PAD), jnp.float32),
        scratch_types=[
            pltpu.VMEM((2, 8, 8, CCOLS), jnp.float32),
            pltpu.VMEM((HCOLS, D_PAD + 1), jnp.float32),
            pltpu.SemaphoreType.DMA((2,)),
        ],
        compiler_params=pltpu.CompilerParams(needs_layout_passes=False),
    )
    def retile_kernel(tblT_hbm, big_hbm, in_v, out_v, sems):
        wid = lax.axis_index("s") * NUM_CORES + lax.axis_index("c")
        col0 = wid * (MAIN_CHUNKS * CCOLS)

        def load_chunk(buf, cstart):
            for k in range(8):
                pltpu.make_async_copy(
                    tblT_hbm.at[pl.ds(8 * k, 8), pl.ds(cstart, CCOLS)],
                    in_v.at[buf, k],
                    sems.at[buf],
                ).start()

        def wait_chunk(buf, cstart):
            for k in range(8):
                pltpu.make_async_copy(
                    tblT_hbm.at[pl.ds(8 * k, 8), pl.ds(cstart, CCOLS)],
                    in_v.at[buf, k],
                    sems.at[buf],
                ).wait()

        def extract_half(buf, h):
            def cgbody(cg, _):
                coff = pl.multiple_of(LANES * cg, LANES)
                rvec = _iota16() + coff
                for k in range(8):
                    for dd in range(8):
                        d = 8 * k + dd
                        x = in_v[buf, k, dd, pl.ds(coff + h * HCOLS, LANES)]
                        plsc.store_scatter(
                            out_v, [rvec, jnp.full((LANES,), d, jnp.int32)], x
                        )
                return _
            lax.fori_loop(0, HCOLS // LANES, cgbody, None)

        def flush(hstart):
            pltpu.sync_copy(
                out_v.at[:, pl.ds(0, D_PAD)],
                big_hbm.at[pl.ds(pl.multiple_of(hstart, HCOLS), HCOLS)],
            )

        load_chunk(0, col0)

        def step(g, _):
            buf = lax.rem(g, 2)
            cstart = pl.multiple_of(col0 + g * CCOLS, CCOLS)
            wait_chunk(buf, cstart)

            @pl.when(g + 1 < MAIN_CHUNKS)
            def _():
                load_chunk(1 - buf, cstart + CCOLS)

            for h in range(2):
                extract_half(buf, h)
                flush(cstart + h * HCOLS)
            return _

        lax.fori_loop(0, MAIN_CHUNKS, step, None)

        @pl.when(wid >= NW - 2)
        def _tail():
            estart = pl.multiple_of(MAIN_COLS + (wid - (NW - 2)) * CCOLS, CCOLS)
            load_chunk(0, estart)
            wait_chunk(0, estart)
            for h in range(2):
                extract_half(0, h)
                flush(estart + h * HCOLS)

    return retile_kernel


def _make_gather():
    mesh = plsc.VectorSubcoreMesh(core_axis_name="c", subcore_axis_name="s")

    @functools.partial(
        pl.kernel,
        mesh=mesh,
        out_type=jax.ShapeDtypeStruct((TOTAL, D_PAD), jnp.float32),
        scratch_types=[
            pltpu.VMEM((NCHUNK, CHUNK), jnp.int32),
            pltpu.VMEM((NBUF, CHUNK, D_PAD), jnp.float32),
            pltpu.SemaphoreType.DMA((NBUF,)),
        ],
    )
    def gather_kernel(big_hbm, idx_hbm, out_hbm, idx_v, rows_v, sems):
        wid = lax.axis_index("s") * NUM_CORES + lax.axis_index("c")
        base = wid * (NCHUNK * CHUNK)

        pltpu.sync_copy(idx_hbm.at[wid], idx_v)

        for b in range(NBUF):
            pltpu.make_async_copy(
                big_hbm.at[idx_v.at[b]], rows_v.at[b], sems.at[b]
            ).start()

        def step(g, _):
            for b in range(NBUF):
                j = g * NBUF + b
                pltpu.make_async_copy(
                    big_hbm.at[idx_v.at[j]], rows_v.at[b], sems.at[b]
                ).wait()
                pltpu.sync_copy(
                    rows_v.at[b], out_hbm.at[pl.ds(base + j * CHUNK, CHUNK)]
                )
                nxt = j + NBUF

                @pl.when(nxt < NCHUNK)
                def _():
                    pltpu.make_async_copy(
                        big_hbm.at[idx_v.at[nxt]], rows_v.at[b], sems.at[b]
                    ).start()

            return _

        lax.fori_loop(0, NCHUNK // NBUF, step, None)

    return gather_kernel


_retile = _make_retile()
_gather = _make_gather()


def kernel(token_ids, table):
    tblT = table.T
    idx3 = token_ids.astype(jnp.int32).reshape(NW, NCHUNK, CHUNK)
    big = _retile(tblT)
    tail = jnp.pad(table[MINI_START:], ((0, 0), (0, D_PAD - D_MODEL)))
    big = lax.dynamic_update_slice(big, tail, (MINI_START, 0))
    out = _gather(big, idx3)
    return out[:, :D_MODEL].reshape(BATCH, SEQ, D_MODEL)


# speedup vs baseline: 1.0028x; 1.0023x over previous
import functools

import jax
import jax.numpy as jnp
from jax import lax
from jax.experimental import pallas as pl
from jax.experimental.pallas import tpu as pltpu
from jax.experimental.pallas import tpu_sc as plsc

NUM_CORES = 2
NUM_SUBCORES = 16
NW = NUM_CORES * NUM_SUBCORES

D_MODEL = 64
D_PAD = 128
VOCAB = 1000000
BATCH = 4096
SEQ = 50
TOTAL = BATCH * SEQ

LANES = 16

CCOLS = 512
HCOLS = CCOLS // 2
MAIN_CHUNKS = 61
MAIN_COLS = MAIN_CHUNKS * CCOLS * NW
MINI_START = MAIN_COLS + CCOLS
OUT_WORDS = HCOLS * D_PAD

CHUNK = 128
NCHUNK = TOTAL // NW // CHUNK
NBUF = 5


def _iota16():
    return jnp.arange(LANES, dtype=jnp.int32)


def _make_retile():
    mesh = plsc.VectorSubcoreMesh(core_axis_name="c", subcore_axis_name="s")

    @functools.partial(
        pl.kernel,
        mesh=mesh,
        out_type=jax.ShapeDtypeStruct((VOCAB * D_PAD,), jnp.float32),
        scratch_types=[
            pltpu.VMEM((2, 8, 8, CCOLS), jnp.float32),
            pltpu.VMEM((HCOLS * D_PAD,), jnp.float32),
            pltpu.VMEM((HCOLS,), jnp.int32),
            pltpu.SemaphoreType.DMA((2,)),
        ],
        compiler_params=pltpu.CompilerParams(needs_layout_passes=False),
    )
    def retile_kernel(tblT_hbm, big_hbm, in_v, out_v, fvec_v, sems):
        wid = lax.axis_index("s") * NUM_CORES + lax.axis_index("c")
        col0 = wid * (MAIN_CHUNKS * CCOLS)

        for cg in range(HCOLS // LANES):
            c = _iota16() + (LANES * cg)
            fvec_v[pl.ds(LANES * cg, LANES)] = c << 7

        def load_chunk(buf, cstart):
            for k in range(8):
                pltpu.make_async_copy(
                    tblT_hbm.at[pl.ds(8 * k, 8), pl.ds(cstart, CCOLS)],
                    in_v.at[buf, k],
                    sems.at[buf],
                ).start()

        def wait_chunk(buf, cstart):
            for k in range(8):
                pltpu.make_async_copy(
                    tblT_hbm.at[pl.ds(8 * k, 8), pl.ds(cstart, CCOLS)],
                    in_v.at[buf, k],
                    sems.at[buf],
                ).wait()

        def extract_half(buf, h):
            def cgbody(cg, _):
                coff = pl.multiple_of(LANES * cg, LANES)
                fv = fvec_v[pl.ds(coff, LANES)]
                for k in range(8):
                    for dd in range(8):
                        x = in_v[buf, k, dd, pl.ds(coff + h * HCOLS, LANES)]
                        plsc.store_scatter(out_v, [fv + (8 * k + dd)], x)
                return _
            lax.fori_loop(0, HCOLS // LANES, cgbody, None)

        def flush(hstart):
            pltpu.sync_copy(
                out_v,
                big_hbm.at[pl.ds(pl.multiple_of(hstart * D_PAD, HCOLS * D_PAD),
                                 HCOLS * D_PAD)],
            )

        load_chunk(0, col0)

        def step(g, _):
            buf = lax.rem(g, 2)
            cstart = pl.multiple_of(col0 + g * CCOLS, CCOLS)
            wait_chunk(buf, cstart)

            @pl.when(g + 1 < MAIN_CHUNKS)
            def _():
                load_chunk(1 - buf, cstart + CCOLS)

            for h in range(2):
                extract_half(buf, h)
                flush(cstart + h * HCOLS)
            return _

        lax.fori_loop(0, MAIN_CHUNKS, step, None)

        @pl.when(wid == NW - 1)
        def _tail():
            load_chunk(0, MAIN_COLS)
            wait_chunk(0, MAIN_COLS)
            for h in range(2):
                extract_half(0, h)
                flush(MAIN_COLS + h * HCOLS)

    return retile_kernel


def _make_gather():
    mesh = plsc.VectorSubcoreMesh(core_axis_name="c", subcore_axis_name="s")

    @functools.partial(
        pl.kernel,
        mesh=mesh,
        out_type=jax.ShapeDtypeStruct((TOTAL, D_PAD), jnp.float32),
        scratch_types=[
            pltpu.VMEM((NCHUNK, CHUNK), jnp.int32),
            pltpu.VMEM((NBUF, CHUNK, D_PAD), jnp.float32),
            pltpu.SemaphoreType.DMA((NBUF,)),
        ],
    )
    def gather_kernel(big_hbm, idx_hbm, out_hbm, idx_v, rows_v, sems):
        wid = lax.axis_index("s") * NUM_CORES + lax.axis_index("c")
        base = wid * (NCHUNK * CHUNK)

        pltpu.sync_copy(idx_hbm.at[wid], idx_v)

        for b in range(NBUF):
            pltpu.make_async_copy(
                big_hbm.at[idx_v.at[b]], rows_v.at[b], sems.at[b]
            ).start()

        def step(g, _):
            for b in range(NBUF):
                j = g * NBUF + b
                pltpu.make_async_copy(
                    big_hbm.at[idx_v.at[j]], rows_v.at[b], sems.at[b]
                ).wait()
                pltpu.sync_copy(
                    rows_v.at[b], out_hbm.at[pl.ds(base + j * CHUNK, CHUNK)]
                )
                nxt = j + NBUF

                @pl.when(nxt < NCHUNK)
                def _():
                    pltpu.make_async_copy(
                        big_hbm.at[idx_v.at[nxt]], rows_v.at[b], sems.at[b]
                    ).start()

            return _

        lax.fori_loop(0, NCHUNK // NBUF, step, None)

    return gather_kernel


_retile = _make_retile()
_gather = _make_gather()


def kernel(token_ids, table):
    tblT = table.T
    idx3 = token_ids.astype(jnp.int32).reshape(NW, NCHUNK, CHUNK)
    big_flat = _retile(tblT)
    tail = jnp.pad(table[MINI_START:], ((0, 0), (0, D_PAD - D_MODEL)))
    big_flat = lax.dynamic_update_slice(
        big_flat, tail.reshape(-1), (MINI_START * D_PAD,)
    )
    big = big_flat.reshape(VOCAB, D_PAD)
    out = _gather(big, idx3)
    return out[:, :D_MODEL].reshape(BATCH, SEQ, D_MODEL)


# speedup vs baseline: 1.2035x; 1.2001x over previous
import functools

import jax
import jax.numpy as jnp
from jax import lax
from jax.experimental import pallas as pl
from jax.experimental.pallas import tpu as pltpu
from jax.experimental.pallas import tpu_sc as plsc

NUM_CORES = 2
NUM_SUBCORES = 16
NW = NUM_CORES * NUM_SUBCORES

D_MODEL = 64
D_PAD = 128
VOCAB = 1000000
BATCH = 4096
SEQ = 50
TOTAL = BATCH * SEQ

LANES = 16

CCOLS = 512
HCOLS = CCOLS // 2
MAIN_CHUNKS = 61
MAIN_COLS = MAIN_CHUNKS * CCOLS * NW
MINI_START = MAIN_COLS + CCOLS
OUT_WORDS = HCOLS * D_PAD

CHUNK = 128
NCHUNK = TOTAL // NW // CHUNK
NBUF = 5


def _iota16():
    return jnp.arange(LANES, dtype=jnp.int32)


def _make_retile():
    mesh = plsc.VectorSubcoreMesh(core_axis_name="c", subcore_axis_name="s")

    @functools.partial(
        pl.kernel,
        mesh=mesh,
        out_type=jax.ShapeDtypeStruct((VOCAB * D_PAD,), jnp.float32),
        scratch_types=[
            pltpu.VMEM((2, 8, 8, CCOLS), jnp.float32),
            pltpu.VMEM((HCOLS * D_PAD,), jnp.float32),
            pltpu.VMEM((HCOLS,), jnp.int32),
            pltpu.SemaphoreType.DMA((2,)),
        ],
        compiler_params=pltpu.CompilerParams(needs_layout_passes=False),
    )
    def retile_kernel(tblT_hbm, big_hbm, in_v, out_v, fvec_v, sems):
        wid = lax.axis_index("s") * NUM_CORES + lax.axis_index("c")
        col0 = wid * (MAIN_CHUNKS * CCOLS)

        for cg in range(HCOLS // LANES):
            c = _iota16() + (LANES * cg)
            fvec_v[pl.ds(LANES * cg, LANES)] = c << 7

        def load_chunk(buf, cstart):
            for k in range(8):
                pltpu.make_async_copy(
                    tblT_hbm.at[pl.ds(8 * k, 8), pl.ds(cstart, CCOLS)],
                    in_v.at[buf, k],
                    sems.at[buf],
                ).start()

        def wait_chunk(buf, cstart):
            for k in range(8):
                pltpu.make_async_copy(
                    tblT_hbm.at[pl.ds(8 * k, 8), pl.ds(cstart, CCOLS)],
                    in_v.at[buf, k],
                    sems.at[buf],
                ).wait()

        def extract_half(buf, h):
            @plsc.parallel_loop(0, HCOLS // LANES, unroll=2)
            def cgbody(cg):
                coff = pl.multiple_of(LANES * cg, LANES)
                fv = fvec_v[pl.ds(coff, LANES)]
                for k in range(8):
                    for dd in range(8):
                        x = in_v[buf, k, dd, pl.ds(coff + h * HCOLS, LANES)]
                        plsc.store_scatter(out_v, [fv + (8 * k + dd)], x)

        def flush(hstart):
            pltpu.sync_copy(
                out_v,
                big_hbm.at[pl.ds(pl.multiple_of(hstart * D_PAD, HCOLS * D_PAD),
                                 HCOLS * D_PAD)],
            )

        load_chunk(0, col0)

        def step(g, _):
            buf = lax.rem(g, 2)
            cstart = pl.multiple_of(col0 + g * CCOLS, CCOLS)
            wait_chunk(buf, cstart)

            @pl.when(g + 1 < MAIN_CHUNKS)
            def _():
                load_chunk(1 - buf, cstart + CCOLS)

            for h in range(2):
                extract_half(buf, h)
                flush(cstart + h * HCOLS)
            return _

        lax.fori_loop(0, MAIN_CHUNKS, step, None)

        @pl.when(wid == NW - 1)
        def _tail():
            load_chunk(0, MAIN_COLS)
            wait_chunk(0, MAIN_COLS)
            for h in range(2):
                extract_half(0, h)
                flush(MAIN_COLS + h * HCOLS)

    return retile_kernel


def _make_gather():
    mesh = plsc.VectorSubcoreMesh(core_axis_name="c", subcore_axis_name="s")

    @functools.partial(
        pl.kernel,
        mesh=mesh,
        out_type=jax.ShapeDtypeStruct((TOTAL, D_PAD), jnp.float32),
        scratch_types=[
            pltpu.VMEM((NCHUNK, CHUNK), jnp.int32),
            pltpu.VMEM((NBUF, CHUNK, D_PAD), jnp.float32),
            pltpu.SemaphoreType.DMA((NBUF,)),
        ],
    )
    def gather_kernel(big_hbm, idx_hbm, out_hbm, idx_v, rows_v, sems):
        wid = lax.axis_index("s") * NUM_CORES + lax.axis_index("c")
        base = wid * (NCHUNK * CHUNK)

        pltpu.sync_copy(idx_hbm.at[wid], idx_v)

        for b in range(NBUF):
            pltpu.make_async_copy(
                big_hbm.at[idx_v.at[b]], rows_v.at[b], sems.at[b]
            ).start()

        def step(g, _):
            for b in range(NBUF):
                j = g * NBUF + b
                pltpu.make_async_copy(
                    big_hbm.at[idx_v.at[j]], rows_v.at[b], sems.at[b]
                ).wait()
                pltpu.sync_copy(
                    rows_v.at[b], out_hbm.at[pl.ds(base + j * CHUNK, CHUNK)]
                )
                nxt = j + NBUF

                @pl.when(nxt < NCHUNK)
                def _():
                    pltpu.make_async_copy(
                        big_hbm.at[idx_v.at[nxt]], rows_v.at[b], sems.at[b]
                    ).start()

            return _

        lax.fori_loop(0, NCHUNK // NBUF, step, None)

    return gather_kernel


_retile = _make_retile()
_gather = _make_gather()


def kernel(token_ids, table):
    tblT = table.T
    idx3 = token_ids.astype(jnp.int32).reshape(NW, NCHUNK, CHUNK)
    big_flat = _retile(tblT)
    tail = jnp.pad(table[MINI_START:], ((0, 0), (0, D_PAD - D_MODEL)))
    big_flat = lax.dynamic_update_slice(
        big_flat, tail.reshape(-1), (MINI_START * D_PAD,)
    )
    big = big_flat.reshape(VOCAB, D_PAD)
    out = _gather(big, idx3)
    return out[:, :D_MODEL].reshape(BATCH, SEQ, D_MODEL)


# speedup vs baseline: 1.2611x; 1.0479x over previous
import functools

import jax
import jax.numpy as jnp
from jax import lax
from jax.experimental import pallas as pl
from jax.experimental.pallas import tpu as pltpu
from jax.experimental.pallas import tpu_sc as plsc

NUM_CORES = 2
NUM_SUBCORES = 16
NW = NUM_CORES * NUM_SUBCORES

D_MODEL = 64
D_PAD = 128
VOCAB = 1000000
BATCH = 4096
SEQ = 50
TOTAL = BATCH * SEQ

LANES = 16

CCOLS = 512
HCOLS = CCOLS // 2
MAIN_CHUNKS = 61
MAIN_COLS = MAIN_CHUNKS * CCOLS * NW
MINI_START = MAIN_COLS + CCOLS
OUT_WORDS = HCOLS * D_PAD

CHUNK = 128
NCHUNK = TOTAL // NW // CHUNK
NBUF = 5


def _iota16():
    return jnp.arange(LANES, dtype=jnp.int32)


def _make_retile():
    mesh = plsc.VectorSubcoreMesh(core_axis_name="c", subcore_axis_name="s")

    @functools.partial(
        pl.kernel,
        mesh=mesh,
        out_type=jax.ShapeDtypeStruct((VOCAB * D_PAD,), jnp.float32),
        scratch_types=[
            pltpu.VMEM((2, 8, 8, CCOLS), jnp.float32),
            pltpu.VMEM((HCOLS * D_PAD,), jnp.float32),
            pltpu.VMEM((HCOLS,), jnp.int32),
            pltpu.SemaphoreType.DMA((2,)),
        ],
        compiler_params=pltpu.CompilerParams(needs_layout_passes=False),
    )
    def retile_kernel(tblT_hbm, big_hbm, in_v, out_v, fvec_v, sems):
        wid = lax.axis_index("s") * NUM_CORES + lax.axis_index("c")
        col0 = wid * (MAIN_CHUNKS * CCOLS)

        for cg in range(HCOLS // LANES):
            c = _iota16() + (LANES * cg)
            fvec_v[pl.ds(LANES * cg, LANES)] = c << 7

        def load_chunk(buf, cstart):
            for k in range(8):
                pltpu.make_async_copy(
                    tblT_hbm.at[pl.ds(8 * k, 8), pl.ds(cstart, CCOLS)],
                    in_v.at[buf, k],
                    sems.at[buf],
                ).start()

        def wait_chunk(buf, cstart):
            for k in range(8):
                pltpu.make_async_copy(
                    tblT_hbm.at[pl.ds(8 * k, 8), pl.ds(cstart, CCOLS)],
                    in_v.at[buf, k],
                    sems.at[buf],
                ).wait()

        def extract_half(buf, h):
            @plsc.parallel_loop(0, HCOLS // LANES, unroll=4)
            def cgbody(cg):
                coff = pl.multiple_of(LANES * cg, LANES)
                fv = fvec_v[pl.ds(coff, LANES)]
                for k in range(8):
                    xs = [
                        in_v[buf, k, dd, pl.ds(coff + h * HCOLS, LANES)]
                        for dd in range(8)
                    ]
                    for dd in range(8):
                        plsc.store_scatter(out_v, [fv + (8 * k + dd)], xs[dd])

        def flush(hstart):
            pltpu.sync_copy(
                out_v,
                big_hbm.at[pl.ds(pl.multiple_of(hstart * D_PAD, HCOLS * D_PAD),
                                 HCOLS * D_PAD)],
            )

        load_chunk(0, col0)

        def step(g, _):
            buf = lax.rem(g, 2)
            cstart = pl.multiple_of(col0 + g * CCOLS, CCOLS)
            wait_chunk(buf, cstart)

            @pl.when(g + 1 < MAIN_CHUNKS)
            def _():
                load_chunk(1 - buf, cstart + CCOLS)

            for h in range(2):
                extract_half(buf, h)
                flush(cstart + h * HCOLS)
            return _

        lax.fori_loop(0, MAIN_CHUNKS, step, None)

        @pl.when(wid == NW - 1)
        def _tail():
            load_chunk(0, MAIN_COLS)
            wait_chunk(0, MAIN_COLS)
            for h in range(2):
                extract_half(0, h)
                flush(MAIN_COLS + h * HCOLS)

    return retile_kernel


def _make_gather():
    mesh = plsc.VectorSubcoreMesh(core_axis_name="c", subcore_axis_name="s")

    @functools.partial(
        pl.kernel,
        mesh=mesh,
        out_type=jax.ShapeDtypeStruct((TOTAL, D_PAD), jnp.float32),
        scratch_types=[
            pltpu.VMEM((NCHUNK, CHUNK), jnp.int32),
            pltpu.VMEM((NBUF, CHUNK, D_PAD), jnp.float32),
            pltpu.SemaphoreType.DMA((NBUF,)),
        ],
    )
    def gather_kernel(big_hbm, idx_hbm, out_hbm, idx_v, rows_v, sems):
        wid = lax.axis_index("s") * NUM_CORES + lax.axis_index("c")
        base = wid * (NCHUNK * CHUNK)

        pltpu.sync_copy(idx_hbm.at[wid], idx_v)

        for b in range(NBUF):
            pltpu.make_async_copy(
                big_hbm.at[idx_v.at[b]], rows_v.at[b], sems.at[b]
            ).start()

        def step(g, _):
            for b in range(NBUF):
                j = g * NBUF + b
                pltpu.make_async_copy(
                    big_hbm.at[idx_v.at[j]], rows_v.at[b], sems.at[b]
                ).wait()
                pltpu.sync_copy(
                    rows_v.at[b], out_hbm.at[pl.ds(base + j * CHUNK, CHUNK)]
                )
                nxt = j + NBUF

                @pl.when(nxt < NCHUNK)
                def _():
                    pltpu.make_async_copy(
                        big_hbm.at[idx_v.at[nxt]], rows_v.at[b], sems.at[b]
                    ).start()

            return _

        lax.fori_loop(0, NCHUNK // NBUF, step, None)

    return gather_kernel


_retile = _make_retile()
_gather = _make_gather()


def kernel(token_ids, table):
    tblT = table.T
    idx3 = token_ids.astype(jnp.int32).reshape(NW, NCHUNK, CHUNK)
    big_flat = _retile(tblT)
    tail = jnp.pad(table[MINI_START:], ((0, 0), (0, D_PAD - D_MODEL)))
    big_flat = lax.dynamic_update_slice(
        big_flat, tail.reshape(-1), (MINI_START * D_PAD,)
    )
    big = big_flat.reshape(VOCAB, D_PAD)
    out = _gather(big, idx3)
    return out[:, :D_MODEL].reshape(BATCH, SEQ, D_MODEL)


# speedup vs baseline: 2.3941x; 1.8984x over previous
import functools

import jax
import jax.numpy as jnp
from jax import lax
from jax.experimental import pallas as pl
from jax.experimental.pallas import tpu as pltpu
from jax.experimental.pallas import tpu_sc as plsc

NUM_CORES = 2
NUM_SUBCORES = 16
NW = NUM_CORES * NUM_SUBCORES

D_MODEL = 64
D_PAD = 128
VOCAB = 1000000
BATCH = 4096
SEQ = 50
SEQ_PAD = 56

BPW = BATCH // NW
NBUF = 8


def _make_gather():
    mesh = plsc.VectorSubcoreMesh(core_axis_name="c", subcore_axis_name="s")

    @functools.partial(
        pl.kernel,
        mesh=mesh,
        out_type=jax.ShapeDtypeStruct((BATCH, SEQ_PAD, D_PAD), jnp.float32),
        scratch_types=[
            pltpu.VMEM((BPW, SEQ_PAD), jnp.int32),
            pltpu.VMEM((NBUF, SEQ_PAD, D_PAD), jnp.float32),
            pltpu.SemaphoreType.DMA((NBUF,)),
        ],
    )
    def gather_kernel(table_hbm, idx_hbm, out_hbm, idx_v, rows_v, sems):
        wid = lax.axis_index("s") * NUM_CORES + lax.axis_index("c")
        gb0 = wid * BPW

        pltpu.sync_copy(idx_hbm.at[wid], idx_v)

        for b in range(NBUF):
            pltpu.make_async_copy(
                table_hbm.at[idx_v.at[b]], rows_v.at[b], sems.at[b]
            ).start()

        def step(g, _):
            for b in range(NBUF):
                j = g * NBUF + b
                pltpu.make_async_copy(
                    table_hbm.at[idx_v.at[j]], rows_v.at[b], sems.at[b]
                ).wait()
                pltpu.sync_copy(rows_v.at[b], out_hbm.at[gb0 + j])
                nxt = j + NBUF

                @pl.when(nxt < BPW)
                def _():
                    pltpu.make_async_copy(
                        table_hbm.at[idx_v.at[nxt]], rows_v.at[b], sems.at[b]
                    ).start()

            return _

        lax.fori_loop(0, BPW // NBUF, step, None)

    return gather_kernel


_gather = _make_gather()


def kernel(token_ids, table):
    tbl128 = jnp.pad(table, ((0, 0), (0, D_PAD - D_MODEL)))
    idxp = jnp.pad(
        token_ids.astype(jnp.int32), ((0, 0), (0, SEQ_PAD - SEQ)), mode="edge"
    )
    idx3 = idxp.reshape(NW, BPW, SEQ_PAD)
    out4 = _gather(tbl128, idx3)
    return out4[:, :SEQ, :D_MODEL]
